# baseline (speedup 1.0000x reference)
import jax
import jax.numpy as jnp
from jax import lax
from jax.experimental import pallas as pl
from jax.experimental.pallas import tpu as pltpu
from jax.experimental.pallas import tpu_sc as plsc

N = 10000
E = 320000
D = 128
H = 64
NC = 2
NS = 16
NP = 10240
SLAB = NP // NS
RP = 2560
ROWS_PER_TILE = RP // NS
WIN = 2
NWIN = ROWS_PER_TILE // WIN
NBUF = 4
CHUNK_W = 20
NCHUNK = NWIN // CHUNK_W
NBLK = 1024
GRID = NP // NBLK

_mesh = plsc.VectorSubcoreMesh(
    core_axis_name="c", subcore_axis_name="s", num_cores=NC, num_subcores=NS)
_sc_params = pltpu.CompilerParams(use_tc_tiling_on_sc=False)


def _deg_body(dst_h, zeros_h, ones_h, deg_out, deg_sh, idx_v, ones_v, zbuf_v,
              bcast_v, sem):
    c = lax.axis_index("c")
    s = lax.axis_index("s")
    pltpu.sync_copy(zeros_h.at[pl.ds(s * SLAB, SLAB)], zbuf_v)
    pltpu.sync_copy(zbuf_v, deg_sh.at[pl.ds(s * SLAB, SLAB)])
    pltpu.sync_copy(ones_h, ones_v)
    pltpu.sync_copy(dst_h.at[pl.ds(s * ROWS_PER_TILE, ROWS_PER_TILE)], idx_v)
    plsc.subcore_barrier()

    def body(w, carry):
        descs = [
            pltpu.async_copy(ones_v, deg_sh.at[idx_v.at[w * 8 + j]], sem,
                             add=True)
            for j in range(8)
        ]
        for d in descs:
            d.wait()
        return carry

    lax.fori_loop(0, ROWS_PER_TILE // 8, body, 0)
    plsc.subcore_barrier()
    npc = NP // NC // NS
    n0 = c * (NP // NC) + s * npc
    pltpu.sync_copy(deg_sh.at[pl.ds(n0, npc)], zbuf_v.at[pl.ds(0, npc)])

    def bbody(g, carry):
        vals = zbuf_v[pl.ds(g * 16, 16)]
        for i in range(16):
            vec = jnp.broadcast_to(vals[i], (16,))
            for k in range(8):
                bcast_v[g * 16 + i, pl.ds(k * 16, 16)] = vec
        return carry

    lax.fori_loop(0, npc // 16, bbody, 0)
    pltpu.sync_copy(bcast_v, deg_out.at[pl.ds(n0, npc)])


def _layer_body(g2_h, e_h, s_out, accum_sh, *scr):
    bufs = scr[0:NBUF]
    es = scr[NBUF:NBUF + 2]
    ed = scr[NBUF + 2:NBUF + 4]
    stg_idx = scr[NBUF + 4]
    sem_g = scr[NBUF + 5:2 * NBUF + 5]
    sem_s = scr[2 * NBUF + 5:3 * NBUF + 5]
    sem_e = scr[3 * NBUF + 5:3 * NBUF + 7]
    c = lax.axis_index("c")
    s = lax.axis_index("s")
    iota2 = lax.iota(jnp.int32, 16) * 2
    base = s * ROWS_PER_TILE
    crows = CHUNK_W * WIN

    def fire_echunk(ch, p):
        row = base + ch * crows
        pltpu.async_copy(e_h.at[0, pl.ds(row, crows)], es[p], sem_e[p])
        pltpu.async_copy(e_h.at[1, pl.ds(row, crows)], ed[p], sem_e[p])

    def wait_echunk(p):
        for _ in range(2):
            pltpu.make_async_copy(e_h.at[0, pl.ds(0, crows)], es[p],
                                  sem_e[p]).wait()

    fire_echunk(0, 0)

    def chunk_idx(k):
        for kk in range(8):
            b0 = 2 * (s * SLAB + k * 128 + kk * 16) + c
            stg_idx[k, pl.ds(kk * 16, 16)] = iota2 + b0

    def wait_chunk(sem, q):
        pltpu.make_async_copy(g2_h.at[pl.ds(0, 128)],
                              bufs[q].at[pl.ds(0, 128)], sem).wait()

    for k in range(5):
        q = k % NBUF
        if k >= NBUF:
            wait_chunk(sem_g[0], 0)
            pltpu.sync_copy(bufs[0].at[pl.ds(0, 128)],
                            accum_sh.at[pl.ds(s * SLAB, 128)])
        chunk_idx(k)
        pltpu.async_copy(g2_h.at[stg_idx.at[k]],
                         bufs[q].at[pl.ds(0, 128)], sem_g[q])
    for k in range(1, 5):
        q = k % NBUF
        wait_chunk(sem_g[q], q)
        pltpu.sync_copy(bufs[q].at[pl.ds(0, 128)],
                        accum_sh.at[pl.ds(s * SLAB + k * 128, 128)])
    plsc.subcore_barrier()

    def fire_g(q, es_ref, lrow):
        for j in range(WIN):
            pltpu.async_copy(g2_h.at[es_ref.at[lrow + j]],
                             bufs[q].at[pl.ds(j * 128, 128)], sem_g[q])

    def fire_s(q, ed_ref, lrow):
        for j in range(WIN):
            pltpu.async_copy(bufs[q].at[pl.ds(j * 128, 128)],
                             accum_sh.at[ed_ref.at[lrow + j]], sem_s[q],
                             add=True)

    def wait_win(sem, q):
        pltpu.make_async_copy(g2_h.at[pl.ds(0, WIN * 128)], bufs[q],
                              sem).wait()

    def win_body(i, ch, p, first):
        for q in range(NBUF):
            lw = i * NBUF + q
            if not first:
                wait_win(sem_s[q], q)
            fire_g(q, es[p], lw * WIN)
            qs = (q - 2) % NBUF
            if first and q < 2:
                continue
            wait_win(sem_g[qs], qs)
            if isinstance(i, int) and i == 0 and q < 2:
                fire_s(qs, ed[1 - p], (crows - 4 + q * WIN))
            else:
                fire_s(qs, ed[p], (lw - 2) * WIN)

    for ch in range(NCHUNK):
        p = ch % 2
        wait_echunk(p)

        def tbody(r, carry):
            for k in range(8):
                v = es[p][r, pl.ds(k * 16, 16)]
                es[p][r, pl.ds(k * 16, 16)] = v * 2 + c
            return carry

        lax.fori_loop(0, crows, tbody, 0)
        win_body(0, ch, p, first=(ch == 0))
        if ch + 1 < NCHUNK:
            fire_echunk(ch + 1, 1 - p)

        def ibody(i, carry, ch=ch, p=p):
            win_body(i, ch, p, first=False)
            return carry

        lax.fori_loop(1, CHUNK_W // NBUF, ibody, 0)

    pl_last = (NCHUNK - 1) % 2
    wait_win(sem_g[2], 2)
    fire_s(2, ed[pl_last], crows - 4)
    wait_win(sem_g[3], 3)
    fire_s(3, ed[pl_last], crows - 2)
    for q in range(NBUF):
        wait_win(sem_s[q], q)
    plsc.subcore_barrier()
    for k in range(5):
        q = k % NBUF
        if k >= NBUF:
            wait_chunk(sem_s[0], 0)
        pltpu.sync_copy(accum_sh.at[pl.ds(s * SLAB + k * 128, 128)],
                        bufs[q].at[pl.ds(0, 128)])
        chunk_idx(k)
        pltpu.async_copy(bufs[q].at[pl.ds(0, 128)],
                         s_out.at[stg_idx.at[k]], sem_s[q])
    for k in range(1, 5):
        wait_chunk(sem_s[k % NBUF], k % NBUF)


_deg_kernel = pl.kernel(
    _deg_body,
    out_type=jax.ShapeDtypeStruct((NP, D), jnp.float32),
    mesh=_mesh,
    scratch_types=[
        pltpu.VMEM_SHARED((NP,), jnp.float32),
        pltpu.VMEM((ROWS_PER_TILE, 128), jnp.int32),
        pltpu.VMEM((128,), jnp.float32),
        pltpu.VMEM((SLAB,), jnp.float32),
        pltpu.VMEM((NP // NC // NS, D), jnp.float32),
        pltpu.SemaphoreType.DMA,
    ],
    compiler_params=_sc_params,
)

_layer_kernel = pl.kernel(
    _layer_body,
    out_type=jax.ShapeDtypeStruct((2 * NP, H), jnp.float32),
    mesh=_mesh,
    scratch_types=(
        [pltpu.VMEM_SHARED((NP, H), jnp.float32)]
        + [pltpu.VMEM((WIN * 128, H), jnp.float32) for _ in range(NBUF)]
        + [pltpu.VMEM((CHUNK_W * WIN, 128), jnp.int32) for _ in range(4)]
        + [pltpu.VMEM((5, 128), jnp.int32)]
        + [pltpu.SemaphoreType.DMA for _ in range(2 * NBUF + 2)]
    ),
    compiler_params=_sc_params,
)


def _mm_body(x_ref, w_ref, h_ref):
    h_ref[...] = jnp.dot(x_ref[...], w_ref[...],
                         preferred_element_type=jnp.float32)


def _scale_body(h_ref, deg_ref, g_ref):
    dinv = lax.rsqrt(deg_ref[...] + 1.0)
    g_ref[...] = h_ref[...] * dinv


def _mid_body(s_ref, deg_ref, b_ref, w_ref, g_ref):
    dinv = lax.rsqrt(deg_ref[...] + 1.0)
    a = jnp.maximum(dinv * s_ref[...] + b_ref[...], 0.0)
    h = jnp.dot(a, w_ref[...], preferred_element_type=jnp.float32)
    g_ref[...] = h * dinv


def _final_body(s_ref, deg_ref, b_ref, out_ref):
    dinv = lax.rsqrt(deg_ref[...] + 1.0)
    out_ref[...] = jnp.maximum(dinv * s_ref[...] + b_ref[...], 0.0)


_mm_call = pl.pallas_call(
    _mm_body,
    grid=(GRID,),
    in_specs=[
        pl.BlockSpec((NBLK, D), lambda i: (i, 0)),
        pl.BlockSpec((D, D), lambda i: (0, 0)),
    ],
    out_specs=pl.BlockSpec((NBLK, D), lambda i: (i, 0)),
    out_shape=jax.ShapeDtypeStruct((NP, D), jnp.float32),
)

_scale_call = pl.pallas_call(
    _scale_body,
    grid=(GRID,),
    in_specs=[
        pl.BlockSpec((NBLK, D), lambda i: (i, 0)),
        pl.BlockSpec((NBLK, D), lambda i: (i, 0)),
    ],
    out_specs=pl.BlockSpec((NBLK, D), lambda i: (i, 0)),
    out_shape=jax.ShapeDtypeStruct((NP, D), jnp.float32),
)

_mid_call = pl.pallas_call(
    _mid_body,
    grid=(GRID,),
    in_specs=[
        pl.BlockSpec((NBLK, D), lambda i: (i, 0)),
        pl.BlockSpec((NBLK, D), lambda i: (i, 0)),
        pl.BlockSpec((1, D), lambda i: (0, 0)),
        pl.BlockSpec((D, D), lambda i: (0, 0)),
    ],
    out_specs=pl.BlockSpec((NBLK, D), lambda i: (i, 0)),
    out_shape=jax.ShapeDtypeStruct((NP, D), jnp.float32),
)

_final_call = pl.pallas_call(
    _final_body,
    grid=(GRID,),
    in_specs=[
        pl.BlockSpec((NBLK, D), lambda i: (i, 0)),
        pl.BlockSpec((NBLK, D), lambda i: (i, 0)),
        pl.BlockSpec((1, D), lambda i: (0, 0)),
    ],
    out_specs=pl.BlockSpec((NBLK, D), lambda i: (i, 0)),
    out_shape=jax.ShapeDtypeStruct((N, D), jnp.float32),
)


def kernel(x, edge_index, W1, b1, W2, b2):
    ei = edge_index.astype(jnp.int32)
    npad = RP * 128 - E
    pad = N + (jnp.arange(npad, dtype=jnp.int32) % (NP - N))
    e_pad = jnp.concatenate(
        [ei, jnp.stack([pad, pad])], axis=1).reshape(2, RP, 128)

    b1r = b1.reshape(1, D)
    b2r = b2.reshape(1, D)
    zeros_np = jnp.zeros((NP,), jnp.float32)
    ones_128 = jnp.ones((128,), jnp.float32)

    h1 = _mm_call(x, W1)
    deg_b = _deg_kernel(e_pad[1], zeros_np, ones_128)
    g1 = _scale_call(h1, deg_b)
    s1 = _layer_kernel(g1.reshape(2 * NP, H), e_pad)
    g2 = _mid_call(s1.reshape(NP, D), deg_b, b1r, W2)
    s2 = _layer_kernel(g2.reshape(2 * NP, H), e_pad)
    return _final_call(s2.reshape(NP, D), deg_b, b2r)
